# baseline (speedup 1.0000x reference)
import functools

import jax
import jax.numpy as jnp
from jax import lax
from jax.experimental import pallas as pl
from jax.experimental.pallas import tpu as pltpu
from jax.experimental.pallas import tpu_sc as plsc

_K = 32
_NBUF = 3


@functools.cache
def _build(R, C, V, D, NC, NS):
    NW = NC * NS
    B = R * C
    b_per_w = B // NW
    n_chunks = b_per_w // _K
    n_main_groups = n_chunks // _NBUF - 1
    w_per_r = C // b_per_w

    mesh = plsc.VectorSubcoreMesh(core_axis_name="c", subcore_axis_name="s")

    @functools.partial(
        pl.kernel,
        mesh=mesh,
        out_type=jax.ShapeDtypeStruct((R, C, D), jnp.float32),
        scratch_types=[
            pltpu.VMEM((n_chunks, _K), jnp.int32),
            pltpu.VMEM((_NBUF, _K, D), jnp.float32),
            pltpu.SemaphoreType.DMA((_NBUF,)),
            pltpu.SemaphoreType.DMA((_NBUF,)),
        ],
    )
    def emb(idx_hbm, table_hbm, out_hbm, idx_v, rows_v, sem_in, sem_out):
        wid = lax.axis_index("s") * NC + lax.axis_index("c")
        row = wid // w_per_r
        col = (wid % w_per_r) * b_per_w
        pltpu.sync_copy(idx_hbm.at[wid], idx_v)

        def gather(g, b):
            return pltpu.async_copy(
                table_hbm.at[idx_v.at[g]], rows_v.at[b], sem_in.at[b])

        def put(g, b):
            return pltpu.async_copy(
                rows_v.at[b], out_hbm.at[row, pl.ds(col + g * _K, _K)],
                sem_out.at[b])

        def wait_in(g, b):
            pltpu.make_async_copy(
                table_hbm.at[idx_v.at[g]], rows_v.at[b], sem_in.at[b]).wait()

        def wait_put(g, b):
            pltpu.make_async_copy(
                rows_v.at[b], out_hbm.at[row, pl.ds(col + g * _K, _K)],
                sem_out.at[b]).wait()

        def step(g, b, steady, refill):
            wait_in(g, b)
            put(g, b)
            if steady:
                gp = g - (_NBUF - 1)
                wait_put(gp, gp % _NBUF)
                if refill:
                    gather(g + 1, (g + 1) % _NBUF)

        for b in range(_NBUF):
            gather(b, b)
        for g in range(_NBUF):
            step(g, g, steady=(g == _NBUF - 1), refill=True)

        def group(j, _):
            for b in range(_NBUF):
                step(j * _NBUF + b, b, steady=True, refill=True)
            return 0

        lax.fori_loop(1, n_main_groups + 1, group, 0)
        for g in range((n_main_groups + 1) * _NBUF, n_chunks):
            step(g, g % _NBUF, steady=True, refill=(g + 1 < n_chunks))
        for g in range(n_chunks - (_NBUF - 1), n_chunks):
            wait_put(g, g % _NBUF)

    return emb


def kernel(input_ids, embedding_table):
    R, C = input_ids.shape
    V, D = embedding_table.shape
    info = plsc.get_sparse_core_info()
    NC, NS = info.num_cores, info.num_subcores
    NW = NC * NS
    b_per_w = (R * C) // NW
    idx3 = input_ids.reshape(NW, b_per_w // _K, _K).astype(jnp.int32)
    return _build(R, C, V, D, NC, NS)(idx3, embedding_table)


# speedup vs baseline: 1.0473x; 1.0473x over previous
import functools

import jax
import jax.numpy as jnp
from jax import lax
from jax.experimental import pallas as pl
from jax.experimental.pallas import tpu as pltpu
from jax.experimental.pallas import tpu_sc as plsc

_K = 16
_NBUF = 4


@functools.cache
def _build(R, C, V, D, NC, NS):
    NW = NC * NS
    B = R * C
    b_per_w = B // NW
    n_chunks = b_per_w // _K
    n_main_groups = n_chunks // _NBUF - 1
    w_per_r = C // b_per_w

    mesh = plsc.VectorSubcoreMesh(core_axis_name="c", subcore_axis_name="s")

    @functools.partial(
        pl.kernel,
        mesh=mesh,
        out_type=jax.ShapeDtypeStruct((R, C, D), jnp.float32),
        scratch_types=[
            pltpu.VMEM((b_per_w,), jnp.int32),
            pltpu.VMEM((_NBUF, _K, D), jnp.float32),
            pltpu.SemaphoreType.DMA((_NBUF,)),
            pltpu.SemaphoreType.DMA((_NBUF,)),
        ],
    )
    def emb(idx_hbm, table_hbm, out_hbm, idx_v, rows_v, sem_in, sem_out):
        wid = lax.axis_index("s") * NC + lax.axis_index("c")
        row = wid // w_per_r
        col = (wid % w_per_r) * b_per_w
        pltpu.sync_copy(idx_hbm.at[row, pl.ds(col, b_per_w)], idx_v)

        def ids(g):
            return idx_v[pl.ds(g * _K, _K)]

        def gather(g, b):
            return pltpu.async_copy(
                table_hbm.at[ids(g)], rows_v.at[b], sem_in.at[b])

        def put(g, b):
            return pltpu.async_copy(
                rows_v.at[b], out_hbm.at[row, pl.ds(col + g * _K, _K)],
                sem_out.at[b])

        def step(g, b, refill):
            pltpu.make_async_copy(
                table_hbm.at[ids(g)], rows_v.at[b], sem_in.at[b]).wait()
            put(g, b).wait()
            if refill:
                gather(g + _NBUF, b)

        for b in range(_NBUF):
            gather(b, b)

        def group(j, _):
            for b in range(_NBUF):
                step(j * _NBUF + b, b, refill=True)
            return 0

        lax.fori_loop(0, n_main_groups, group, 0)
        for g in range(n_main_groups * _NBUF, n_chunks):
            step(g, g % _NBUF, refill=(g + _NBUF < n_chunks))

    return emb


def kernel(input_ids, embedding_table):
    R, C = input_ids.shape
    V, D = embedding_table.shape
    info = plsc.get_sparse_core_info()
    return _build(R, C, V, D, info.num_cores, info.num_subcores)(
        input_ids.astype(jnp.int32), embedding_table)
